# baseline (speedup 1.0000x reference)
import jax
import jax.numpy as jnp
from jax.experimental import pallas as pl
from jax.experimental.pallas import tpu as pltpu

_EPS2 = 1e-24
_B = 64
_S = 1024
_D = 128
_BPS = 32
_STEPS = _B // _BPS
_NPAIR = _BPS // 2


def _sim_kernel(x_ref, out_ref, acc_ref):
    j = pl.program_id(0)
    ones_rows = jnp.ones((16, _S), jnp.bfloat16)
    eps2 = jnp.bfloat16(_EPS2)

    yt = []
    for k in range(_BPS):
        xt = jnp.transpose(x_ref[k].astype(jnp.bfloat16), (1, 0))
        n2 = jnp.sum(xt * xt, axis=0, keepdims=True)
        inv = jax.lax.rsqrt(jnp.maximum(n2, eps2))
        yt.append(xt * inv)

    for p in range(_NPAIR):
        ypair = jnp.concatenate([yt[2 * p], yt[2 * p + 1]], axis=0)
        yaug = jnp.concatenate([ypair, ones_rows], axis=0)
        g = jax.lax.dot_general(
            yaug, ypair, (((1,), (1,)), ((), ())),
            preferred_element_type=jnp.float32,
        )
        ga = g[:_D, :_D]
        gb = g[_D:2 * _D, _D:2 * _D]
        mrow = g[2 * _D:2 * _D + 1, :]
        q = jnp.concatenate(
            [jnp.sum(ga * ga, axis=0, keepdims=True),
             jnp.sum(gb * gb, axis=0, keepdims=True)], axis=1)
        part = q - 2.0 * (mrow * mrow)
        acc_ref[p:p + 1, :] = jnp.where(
            j == 0, part, acc_ref[p:p + 1, :] + part)

    @pl.when(j == _STEPS - 1)
    def _():
        denom = float(_B) * float(_S) * float(_S)
        total = jnp.sum(acc_ref[...], axis=(0, 1), keepdims=True)[:1, :1]
        out_ref[...] = total * (1.0 / denom) + 1.0


def kernel(X):
    res = pl.pallas_call(
        _sim_kernel,
        grid=(_STEPS,),
        in_specs=[pl.BlockSpec(
            (_BPS, _S, _D), lambda j: (j, 0, 0))],
        out_specs=pl.BlockSpec((1, 1), lambda j: (0, 0)),
        out_shape=jax.ShapeDtypeStruct((1, 1), jnp.float32),
        scratch_shapes=[
            pltpu.VMEM((_NPAIR, 2 * _D), jnp.float32),
        ],
        compiler_params=pltpu.CompilerParams(
            dimension_semantics=("arbitrary",),
        ),
    )(X)
    return jnp.reshape(res, ())


# speedup vs baseline: 1.0522x; 1.0522x over previous
import jax
import jax.numpy as jnp
from jax.experimental import pallas as pl
from jax.experimental.pallas import tpu as pltpu

_EPS2 = 1e-24
_B = 64
_S = 1024
_D = 128
_BPS = 16
_STEPS = _B // _BPS
_NPAIR = _BPS // 2


def _sim_kernel(x0_ref, x1_ref, out_ref, acc_ref):
    j = pl.program_id(0)
    ones_rows = jnp.ones((16, _S), jnp.bfloat16)
    eps2 = jnp.bfloat16(_EPS2)

    yt = []
    for k in range(_BPS):
        src = x0_ref if k < _BPS // 2 else x1_ref
        xt = jnp.transpose(
            src[k % (_BPS // 2)].astype(jnp.bfloat16), (1, 0))
        n2 = jnp.sum(xt * xt, axis=0, keepdims=True)
        inv = jax.lax.rsqrt(jnp.maximum(n2, eps2))
        yt.append(xt * inv)

    for p in range(_NPAIR):
        ypair = jnp.concatenate([yt[2 * p], yt[2 * p + 1]], axis=0)
        yaug = jnp.concatenate([ypair, ones_rows], axis=0)
        g = jax.lax.dot_general(
            yaug, ypair, (((1,), (1,)), ((), ())),
            preferred_element_type=jnp.float32,
        )
        ga = g[:_D, :_D]
        gb = g[_D:2 * _D, _D:2 * _D]
        mrow = g[2 * _D:2 * _D + 1, :]
        q = jnp.concatenate(
            [jnp.sum(ga * ga, axis=0, keepdims=True),
             jnp.sum(gb * gb, axis=0, keepdims=True)], axis=1)
        part = q - 2.0 * (mrow * mrow)
        acc_ref[p:p + 1, :] = jnp.where(
            j == 0, part, acc_ref[p:p + 1, :] + part)

    @pl.when(j == _STEPS - 1)
    def _():
        denom = float(_B) * float(_S) * float(_S)
        total = jnp.sum(acc_ref[...], axis=(0, 1), keepdims=True)[:1, :1]
        out_ref[...] = total * (1.0 / denom) + 1.0


def kernel(X):
    res = pl.pallas_call(
        _sim_kernel,
        grid=(_STEPS,),
        in_specs=[
            pl.BlockSpec((_BPS // 2, _S, _D), lambda j: (2 * j, 0, 0)),
            pl.BlockSpec((_BPS // 2, _S, _D), lambda j: (2 * j + 1, 0, 0)),
        ],
        out_specs=pl.BlockSpec((1, 1), lambda j: (0, 0)),
        out_shape=jax.ShapeDtypeStruct((1, 1), jnp.float32),
        scratch_shapes=[
            pltpu.VMEM((_NPAIR, 2 * _D), jnp.float32),
        ],
        compiler_params=pltpu.CompilerParams(
            dimension_semantics=("arbitrary",),
        ),
    )(X, X)
    return jnp.reshape(res, ())
